# baseline (speedup 1.0000x reference)
import functools

import jax
import jax.numpy as jnp
from jax.experimental import pallas as pl
from jax.experimental.pallas import tpu as pltpu

CHANNEL = 64
SEG = 4
K = 128
LEVELS = 3
D = CHANNEL // SEG
KT = SEG * K


def _mm(a, b, dims):
    return jax.lax.dot_general(
        a, b, (dims, ((), ())), preferred_element_type=jnp.float32)


def _body(x_ref, w_enc, b_enc, w_q, b_q, w_lat, b_lat, w_deq, b_deq,
          w_res, b_res, w_side, b_side, bd_ref, bg_ref, csq_ref,
          out_ref, loss_ref, *, blk):
    i = pl.program_id(0)
    x = x_ref[...]
    loss = jnp.float32(0.0)
    hards = []
    cur = None
    iota_f = jax.lax.broadcasted_iota(jnp.int32, (blk, K), 1).astype(jnp.float32)
    for l in range(LEVELS):
        if l == 0:
            z = _mm(x, w_enc[l], ((0,), (1,)))
        else:
            z = _mm(cur, w_enc[l], ((1,), (1,)))
        z = z + b_enc[l][None, :]
        q = _mm(z, w_q[l], ((1,), (1,))) + b_q[l][None, :]
        cross = _mm(q, bd_ref[l], ((1,), (0,)))
        oh_parts = []
        for s in range(SEG):
            qs = q[:, D * s:D * (s + 1)]
            qsq = jnp.sum(qs * qs, axis=1, keepdims=True)
            ds = (qsq - 2.0 * cross[:, K * s:K * (s + 1)]) \
                + csq_ref[l][None, K * s:K * (s + 1)]
            mn = jnp.min(ds, axis=1, keepdims=True)
            masked = jnp.where(ds == mn, iota_f, jnp.float32(K))
            idx = jnp.min(masked, axis=1, keepdims=True)
            oh_parts.append((iota_f == idx).astype(jnp.float32))
        oh = jnp.concatenate(oh_parts, axis=1)
        hard = _mm(oh, bg_ref[l], ((1,), (0,)))
        df = q - hard
        loss = loss + jnp.sum(df * df)
        hards.append(hard)
        if l < LEVELS - 1:
            cur = _mm(z, w_lat[l], ((1,), (1,))) + b_lat[l][None, :]
    t = (_mm(hards[2], w_deq[2], ((1,), (1,))) + b_deq[2][None, :]
         + b_side[2][None, :])
    y = _mm(t, w_res[2], ((1,), (1,))) + b_res[2][None, :]
    for l in (1, 0):
        t = (_mm(hards[l], w_deq[l], ((1,), (1,))) + b_deq[l][None, :]
             + _mm(y, w_side[l], ((1,), (1,))) + b_side[l][None, :])
        y = _mm(t, w_res[l], ((1,), (1,))) + b_res[l][None, :]
    out_ref[...] = y

    @pl.when(i == 0)
    def _init():
        loss_ref[0, 0] = loss

    @pl.when(i != 0)
    def _acc():
        loss_ref[0, 0] += loss


def kernel(heter_feature_2d, W_enc, b_enc, W_q, b_q, W_lat, b_lat,
           W_deq, b_deq, W_res, b_res, W_side, b_side, codebooks):
    Bq, C, Hq, Wq_ = heter_feature_2d.shape
    n = Bq * Hq * Wq_
    x = heter_feature_2d.reshape(C, n)
    blk = 1024
    grid = n // blk

    eye = jnp.eye(SEG, dtype=jnp.float32)
    cbT = jnp.transpose(codebooks, (0, 1, 3, 2))
    bd = jnp.einsum('lsdk,st->ltdsk', cbT, eye).reshape(LEVELS, C, KT)
    bg = jnp.transpose(bd, (0, 2, 1))
    csq = jnp.sum(codebooks * codebooks, axis=-1).reshape(LEVELS, KT)

    full = lambda shape: pl.BlockSpec(shape, lambda i: (0,) * len(shape))
    out, loss = pl.pallas_call(
        functools.partial(_body, blk=blk),
        grid=(grid,),
        in_specs=[
            pl.BlockSpec((C, blk), lambda i: (0, i)),
            full((LEVELS, C, C)), full((LEVELS, C)),
            full((LEVELS, C, C)), full((LEVELS, C)),
            full((LEVELS, C, C)), full((LEVELS, C)),
            full((LEVELS, C, C)), full((LEVELS, C)),
            full((LEVELS, C, C)), full((LEVELS, C)),
            full((LEVELS, C, C)), full((LEVELS, C)),
            full((LEVELS, C, KT)), full((LEVELS, KT, C)),
            full((LEVELS, KT)),
        ],
        out_specs=[
            pl.BlockSpec((blk, C), lambda i: (i, 0)),
            pl.BlockSpec((1, 1), lambda i: (0, 0),
                         memory_space=pltpu.SMEM),
        ],
        out_shape=[
            jax.ShapeDtypeStruct((n, C), jnp.float32),
            jax.ShapeDtypeStruct((1, 1), jnp.float32),
        ],
    )(x, W_enc, b_enc, W_q, b_q, W_lat, b_lat, W_deq, b_deq,
      W_res, b_res, W_side, b_side, bd, bg, csq)

    restored = out.reshape(Bq, Hq, Wq_, C)
    codebook_loss = loss[0, 0] * jnp.float32(1.25) / jnp.float32(n * C)
    return (restored, codebook_loss)


# speedup vs baseline: 1.4742x; 1.4742x over previous
import functools

import jax
import jax.numpy as jnp
from jax.experimental import pallas as pl
from jax.experimental.pallas import tpu as pltpu

CHANNEL = 64
SEG = 4
K = 128
LEVELS = 3
D = CHANNEL // SEG
KT = SEG * K


def _mm(a, b, dims):
    return jax.lax.dot_general(
        a, b, (dims, ((), ())), preferred_element_type=jnp.float32)


def _body(x_ref, w_enc, b_enc, w_q, b_q, w_lat, b_lat, w_deq, b_deq,
          w_res, b_res, w_side, b_side, bd_ref, bg_ref, csq_ref,
          out_ref, loss_ref, *, blk):
    i = pl.program_id(0)
    x = x_ref[...]
    loss = jnp.float32(0.0)
    hards = []
    cur = None
    iota_f = jax.lax.broadcasted_iota(jnp.int32, (blk, K), 1).astype(jnp.float32)
    zs, qs_, crosses = [], [], []
    for l in range(LEVELS):
        if l == 0:
            z = _mm(x, w_enc[l], ((0,), (1,)))
        else:
            z = _mm(cur, w_enc[l], ((1,), (1,)))
        z = z + b_enc[l][None, :]
        q = _mm(z, w_q[l], ((1,), (1,))) + b_q[l][None, :]
        crosses.append(_mm(q, bd_ref[l], ((1,), (0,))))
        zs.append(z)
        qs_.append(q)
        if l < LEVELS - 1:
            cur = _mm(z, w_lat[l], ((1,), (1,))) + b_lat[l][None, :]

    ds_all, mn_all, masked_all, idx_all, oh_all = {}, {}, {}, {}, {}
    for l in range(LEVELS):
        for s in range(SEG):
            qseg = qs_[l][:, D * s:D * (s + 1)]
            qsq = jnp.sum(qseg * qseg, axis=1, keepdims=True)
            ds_all[l, s] = (qsq - 2.0 * crosses[l][:, K * s:K * (s + 1)]) \
                + csq_ref[l][None, K * s:K * (s + 1)]
    for l in range(LEVELS):
        for s in range(SEG):
            mn_all[l, s] = jnp.min(ds_all[l, s], axis=1, keepdims=True)
    for l in range(LEVELS):
        for s in range(SEG):
            masked_all[l, s] = jnp.where(
                ds_all[l, s] == mn_all[l, s], iota_f, jnp.float32(K))
    for l in range(LEVELS):
        for s in range(SEG):
            idx_all[l, s] = jnp.min(masked_all[l, s], axis=1, keepdims=True)
    for l in range(LEVELS):
        for s in range(SEG):
            oh_all[l, s] = (iota_f == idx_all[l, s]).astype(jnp.float32)

    for l in range(LEVELS):
        oh = jnp.concatenate([oh_all[l, s] for s in range(SEG)], axis=1)
        hard = _mm(oh, bg_ref[l], ((1,), (0,)))
        df = qs_[l] - hard
        loss = loss + jnp.sum(df * df)
        hards.append(hard)
    t = (_mm(hards[2], w_deq[2], ((1,), (1,))) + b_deq[2][None, :]
         + b_side[2][None, :])
    y = _mm(t, w_res[2], ((1,), (1,))) + b_res[2][None, :]
    for l in (1, 0):
        t = (_mm(hards[l], w_deq[l], ((1,), (1,))) + b_deq[l][None, :]
             + _mm(y, w_side[l], ((1,), (1,))) + b_side[l][None, :])
        y = _mm(t, w_res[l], ((1,), (1,))) + b_res[l][None, :]
    out_ref[...] = y

    @pl.when(i == 0)
    def _init():
        loss_ref[0, 0] = loss

    @pl.when(i != 0)
    def _acc():
        loss_ref[0, 0] += loss


def kernel(heter_feature_2d, W_enc, b_enc, W_q, b_q, W_lat, b_lat,
           W_deq, b_deq, W_res, b_res, W_side, b_side, codebooks):
    Bq, C, Hq, Wq_ = heter_feature_2d.shape
    n = Bq * Hq * Wq_
    x = heter_feature_2d.reshape(C, n)
    blk = 1024
    grid = n // blk

    eye = jnp.eye(SEG, dtype=jnp.float32)
    cbT = jnp.transpose(codebooks, (0, 1, 3, 2))
    bd = jnp.einsum('lsdk,st->ltdsk', cbT, eye).reshape(LEVELS, C, KT)
    bg = jnp.transpose(bd, (0, 2, 1))
    csq = jnp.sum(codebooks * codebooks, axis=-1).reshape(LEVELS, KT)

    full = lambda shape: pl.BlockSpec(shape, lambda i: (0,) * len(shape))
    out, loss = pl.pallas_call(
        functools.partial(_body, blk=blk),
        grid=(grid,),
        in_specs=[
            pl.BlockSpec((C, blk), lambda i: (0, i)),
            full((LEVELS, C, C)), full((LEVELS, C)),
            full((LEVELS, C, C)), full((LEVELS, C)),
            full((LEVELS, C, C)), full((LEVELS, C)),
            full((LEVELS, C, C)), full((LEVELS, C)),
            full((LEVELS, C, C)), full((LEVELS, C)),
            full((LEVELS, C, C)), full((LEVELS, C)),
            full((LEVELS, C, KT)), full((LEVELS, KT, C)),
            full((LEVELS, KT)),
        ],
        out_specs=[
            pl.BlockSpec((blk, C), lambda i: (i, 0)),
            pl.BlockSpec((1, 1), lambda i: (0, 0),
                         memory_space=pltpu.SMEM),
        ],
        out_shape=[
            jax.ShapeDtypeStruct((n, C), jnp.float32),
            jax.ShapeDtypeStruct((1, 1), jnp.float32),
        ],
    )(x, W_enc, b_enc, W_q, b_q, W_lat, b_lat, W_deq, b_deq,
      W_res, b_res, W_side, b_side, bd, bg, csq)

    restored = out.reshape(Bq, Hq, Wq_, C)
    codebook_loss = loss[0, 0] * jnp.float32(1.25) / jnp.float32(n * C)
    return (restored, codebook_loss)


# speedup vs baseline: 1.8144x; 1.2308x over previous
import functools

import jax
import jax.numpy as jnp
from jax.experimental import pallas as pl
from jax.experimental.pallas import tpu as pltpu

CHANNEL = 64
SEG = 4
K = 128
LEVELS = 3
D = CHANNEL // SEG
KT = SEG * K


def _mm(a, b, dims):
    return jax.lax.dot_general(
        a, b, (dims, ((), ())), preferred_element_type=jnp.float32)


def _body(x_ref, w_enc, b_enc, w_q, b_q, w_lat, b_lat, w_deq, b_deq,
          w_res, b_res, w_side, b_side, bd_ref, bg_ref, csq_ref,
          out_ref, loss_ref, *, blk):
    i = pl.program_id(0)
    x = x_ref[...]
    loss = jnp.float32(0.0)
    hards = []
    cur = None
    iota_f = jax.lax.broadcasted_iota(jnp.int32, (blk, K), 1).astype(jnp.float32)
    zs, qs_, crosses = [], [], []
    for l in range(LEVELS):
        if l == 0:
            z = _mm(x, w_enc[l], ((0,), (1,)))
        else:
            z = _mm(cur, w_enc[l], ((1,), (1,)))
        z = z + b_enc[l][None, :]
        q = _mm(z, w_q[l], ((1,), (1,))) + b_q[l][None, :]
        crosses.append(_mm(q, bd_ref[l], ((1,), (0,))))
        zs.append(z)
        qs_.append(q)
        if l < LEVELS - 1:
            cur = _mm(z, w_lat[l], ((1,), (1,))) + b_lat[l][None, :]

    ds_all, mn_all, masked_all, idx_all, oh_all = {}, {}, {}, {}, {}
    for l in range(LEVELS):
        for s in range(SEG):
            qseg = qs_[l][:, D * s:D * (s + 1)]
            qsq = jnp.sum(qseg * qseg, axis=1, keepdims=True)
            ds_all[l, s] = (qsq - 2.0 * crosses[l][:, K * s:K * (s + 1)]) \
                + csq_ref[l][None, K * s:K * (s + 1)]
    for l in range(LEVELS):
        for s in range(SEG):
            mn_all[l, s] = jnp.min(ds_all[l, s], axis=1, keepdims=True)
    for l in range(LEVELS):
        for s in range(SEG):
            masked_all[l, s] = jnp.where(
                ds_all[l, s] == mn_all[l, s], iota_f, jnp.float32(K))
    for l in range(LEVELS):
        for s in range(SEG):
            idx_all[l, s] = jnp.min(masked_all[l, s], axis=1, keepdims=True)
    for l in range(LEVELS):
        for s in range(SEG):
            oh_all[l, s] = (iota_f == idx_all[l, s]).astype(jnp.float32)

    for l in range(LEVELS):
        oh = jnp.concatenate([oh_all[l, s] for s in range(SEG)], axis=1)
        hard = _mm(oh, bg_ref[l], ((1,), (0,)))
        df = qs_[l] - hard
        loss = loss + jnp.sum(df * df)
        hards.append(hard)
    t = (_mm(hards[2], w_deq[2], ((1,), (1,))) + b_deq[2][None, :]
         + b_side[2][None, :])
    y = _mm(t, w_res[2], ((1,), (1,))) + b_res[2][None, :]
    for l in (1, 0):
        t = (_mm(hards[l], w_deq[l], ((1,), (1,))) + b_deq[l][None, :]
             + _mm(y, w_side[l], ((1,), (1,))) + b_side[l][None, :])
        y = _mm(t, w_res[l], ((1,), (1,))) + b_res[l][None, :]
    out_ref[...] = y

    @pl.when(i == 0)
    def _init():
        loss_ref[0, 0] = loss

    @pl.when(i != 0)
    def _acc():
        loss_ref[0, 0] += loss


def kernel(heter_feature_2d, W_enc, b_enc, W_q, b_q, W_lat, b_lat,
           W_deq, b_deq, W_res, b_res, W_side, b_side, codebooks):
    Bq, C, Hq, Wq_ = heter_feature_2d.shape
    n = Bq * Hq * Wq_
    x = heter_feature_2d.reshape(C, n)
    blk = 2048
    grid = n // blk

    eye = jnp.eye(SEG, dtype=jnp.float32)
    cbT = jnp.transpose(codebooks, (0, 1, 3, 2))
    bd = jnp.einsum('lsdk,st->ltdsk', cbT, eye).reshape(LEVELS, C, KT)
    bg = jnp.transpose(bd, (0, 2, 1))
    csq = jnp.sum(codebooks * codebooks, axis=-1).reshape(LEVELS, KT)

    full = lambda shape: pl.BlockSpec(shape, lambda i: (0,) * len(shape))
    out, loss = pl.pallas_call(
        functools.partial(_body, blk=blk),
        grid=(grid,),
        in_specs=[
            pl.BlockSpec((C, blk), lambda i: (0, i)),
            full((LEVELS, C, C)), full((LEVELS, C)),
            full((LEVELS, C, C)), full((LEVELS, C)),
            full((LEVELS, C, C)), full((LEVELS, C)),
            full((LEVELS, C, C)), full((LEVELS, C)),
            full((LEVELS, C, C)), full((LEVELS, C)),
            full((LEVELS, C, C)), full((LEVELS, C)),
            full((LEVELS, C, KT)), full((LEVELS, KT, C)),
            full((LEVELS, KT)),
        ],
        out_specs=[
            pl.BlockSpec((blk, C), lambda i: (i, 0)),
            pl.BlockSpec((1, 1), lambda i: (0, 0),
                         memory_space=pltpu.SMEM),
        ],
        out_shape=[
            jax.ShapeDtypeStruct((n, C), jnp.float32),
            jax.ShapeDtypeStruct((1, 1), jnp.float32),
        ],
    )(x, W_enc, b_enc, W_q, b_q, W_lat, b_lat, W_deq, b_deq,
      W_res, b_res, W_side, b_side, bd, bg, csq)

    restored = out.reshape(Bq, Hq, Wq_, C)
    codebook_loss = loss[0, 0] * jnp.float32(1.25) / jnp.float32(n * C)
    return (restored, codebook_loss)


# speedup vs baseline: 1.8648x; 1.0278x over previous
import functools

import jax
import jax.numpy as jnp
from jax.experimental import pallas as pl
from jax.experimental.pallas import tpu as pltpu

CHANNEL = 64
SEG = 4
K = 128
LEVELS = 3
D = CHANNEL // SEG
KT = SEG * K


def _mm(a, b, dims):
    return jax.lax.dot_general(
        a, b, (dims, ((), ())), preferred_element_type=jnp.float32)


def _body(x_ref, w_enc, b_enc, w_q, b_q, w_lat, b_lat, w_deq, b_deq,
          w_res, b_res, w_side, b_side, bd_ref, bg_ref, csq_ref,
          out_ref, loss_ref, *, blk):
    i = pl.program_id(0)
    x = x_ref[...]
    loss = jnp.float32(0.0)
    hards = []
    cur = None
    iota_f = jax.lax.broadcasted_iota(jnp.int32, (blk, K), 1).astype(jnp.float32)
    zs, qs_, crosses = [], [], []
    for l in range(LEVELS):
        if l == 0:
            z = _mm(x, w_enc[l], ((0,), (1,)))
        else:
            z = _mm(cur, w_enc[l], ((1,), (1,)))
        z = z + b_enc[l][None, :]
        q = _mm(z, w_q[l], ((1,), (1,))) + b_q[l][None, :]
        crosses.append(_mm(q, bd_ref[l], ((1,), (0,))))
        zs.append(z)
        qs_.append(q)
        if l < LEVELS - 1:
            cur = _mm(z, w_lat[l], ((1,), (1,))) + b_lat[l][None, :]

    pipes = [(l, s) for l in range(LEVELS) for s in range(SEG)]
    sq = [qs_[l] * qs_[l] for l in range(LEVELS)]
    ds_all, mn_all, masked_all, idx_all, oh_all = {}, {}, {}, {}, {}
    for group in (pipes[:6], pipes[6:]):
        for l, s in group:
            qsq = jnp.sum(sq[l][:, D * s:D * (s + 1)], axis=1, keepdims=True)
            ds_all[l, s] = (qsq - 2.0 * crosses[l][:, K * s:K * (s + 1)]) \
                + csq_ref[l][None, K * s:K * (s + 1)]
        for l, s in group:
            mn_all[l, s] = jnp.min(ds_all[l, s], axis=1, keepdims=True)
        for l, s in group:
            masked_all[l, s] = jnp.where(
                ds_all[l, s] == mn_all[l, s], iota_f, jnp.float32(K))
        for l, s in group:
            idx_all[l, s] = jnp.min(masked_all[l, s], axis=1, keepdims=True)
        for l, s in group:
            oh_all[l, s] = (iota_f == idx_all[l, s]).astype(jnp.float32)

    for l in range(LEVELS):
        oh = jnp.concatenate([oh_all[l, s] for s in range(SEG)], axis=1)
        hard = _mm(oh, bg_ref[l], ((1,), (0,)))
        df = qs_[l] - hard
        loss = loss + jnp.sum(df * df)
        hards.append(hard)
    t = (_mm(hards[2], w_deq[2], ((1,), (1,))) + b_deq[2][None, :]
         + b_side[2][None, :])
    y = _mm(t, w_res[2], ((1,), (1,))) + b_res[2][None, :]
    for l in (1, 0):
        t = (_mm(hards[l], w_deq[l], ((1,), (1,))) + b_deq[l][None, :]
             + _mm(y, w_side[l], ((1,), (1,))) + b_side[l][None, :])
        y = _mm(t, w_res[l], ((1,), (1,))) + b_res[l][None, :]
    out_ref[...] = y

    @pl.when(i == 0)
    def _init():
        loss_ref[0, 0] = loss

    @pl.when(i != 0)
    def _acc():
        loss_ref[0, 0] += loss


def kernel(heter_feature_2d, W_enc, b_enc, W_q, b_q, W_lat, b_lat,
           W_deq, b_deq, W_res, b_res, W_side, b_side, codebooks):
    Bq, C, Hq, Wq_ = heter_feature_2d.shape
    n = Bq * Hq * Wq_
    x = heter_feature_2d.reshape(C, n)
    blk = 4096
    grid = n // blk

    eye = jnp.eye(SEG, dtype=jnp.float32)
    cbT = jnp.transpose(codebooks, (0, 1, 3, 2))
    bd = jnp.einsum('lsdk,st->ltdsk', cbT, eye).reshape(LEVELS, C, KT)
    bg = jnp.transpose(bd, (0, 2, 1))
    csq = jnp.sum(codebooks * codebooks, axis=-1).reshape(LEVELS, KT)

    full = lambda shape: pl.BlockSpec(shape, lambda i: (0,) * len(shape))
    out, loss = pl.pallas_call(
        functools.partial(_body, blk=blk),
        grid=(grid,),
        in_specs=[
            pl.BlockSpec((C, blk), lambda i: (0, i)),
            full((LEVELS, C, C)), full((LEVELS, C)),
            full((LEVELS, C, C)), full((LEVELS, C)),
            full((LEVELS, C, C)), full((LEVELS, C)),
            full((LEVELS, C, C)), full((LEVELS, C)),
            full((LEVELS, C, C)), full((LEVELS, C)),
            full((LEVELS, C, C)), full((LEVELS, C)),
            full((LEVELS, C, KT)), full((LEVELS, KT, C)),
            full((LEVELS, KT)),
        ],
        out_specs=[
            pl.BlockSpec((blk, C), lambda i: (i, 0)),
            pl.BlockSpec((1, 1), lambda i: (0, 0),
                         memory_space=pltpu.SMEM),
        ],
        out_shape=[
            jax.ShapeDtypeStruct((n, C), jnp.float32),
            jax.ShapeDtypeStruct((1, 1), jnp.float32),
        ],
    )(x, W_enc, b_enc, W_q, b_q, W_lat, b_lat, W_deq, b_deq,
      W_res, b_res, W_side, b_side, bd, bg, csq)

    restored = out.reshape(Bq, Hq, Wq_, C)
    codebook_loss = loss[0, 0] * jnp.float32(1.25) / jnp.float32(n * C)
    return (restored, codebook_loss)


# speedup vs baseline: 3.7279x; 1.9991x over previous
import functools

import jax
import jax.numpy as jnp
from jax.experimental import pallas as pl
from jax.experimental.pallas import tpu as pltpu

CHANNEL = 64
SEG = 4
K = 128
LEVELS = 3
D = CHANNEL // SEG
KT = SEG * K


def _mm(a, b, dims):
    return jax.lax.dot_general(
        a, b, (dims, ((), ())), preferred_element_type=jnp.float32)


def _body(x_ref, w_enc, b_enc, w_q, b_q, w_lat, b_lat, w_deq, b_deq,
          w_res, b_res, w_side, b_side, bd_ref, bg_ref, csq_ref,
          out_ref, loss_ref, *, blk):
    i = pl.program_id(0)
    cur = x_ref[...]
    iota_f = jax.lax.broadcasted_iota(
        jnp.int32, (K, blk), 0).astype(jnp.float32)

    qs_, crosses = [], []
    for l in range(LEVELS):
        z = _mm(w_enc[l], cur, ((1,), (0,))) + b_enc[l]
        q = _mm(w_q[l], z, ((1,), (0,))) + b_q[l]
        crosses.append(_mm(bd_ref[l], q, ((0,), (0,))))
        qs_.append(q)
        if l < LEVELS - 1:
            cur = _mm(w_lat[l], z, ((1,), (0,))) + b_lat[l]

    pipes = [(l, s) for l in range(LEVELS) for s in range(SEG)]
    sq = [qs_[l] * qs_[l] for l in range(LEVELS)]
    ds_all, mn_all, masked_all, idx_all, oh_all = {}, {}, {}, {}, {}
    for l, s in pipes:
        qsq = jnp.sum(sq[l][D * s:D * (s + 1), :], axis=0, keepdims=True)
        ds_all[l, s] = (qsq - 2.0 * crosses[l][K * s:K * (s + 1), :]) \
            + csq_ref[l][K * s:K * (s + 1), :]
    for l, s in pipes:
        mn_all[l, s] = jnp.min(ds_all[l, s], axis=0, keepdims=True)
    for l, s in pipes:
        masked_all[l, s] = jnp.where(
            ds_all[l, s] == mn_all[l, s], iota_f, jnp.float32(K))
    for l, s in pipes:
        idx_all[l, s] = jnp.min(masked_all[l, s], axis=0, keepdims=True)
    for l, s in pipes:
        oh_all[l, s] = (iota_f == idx_all[l, s]).astype(jnp.float32)

    loss = jnp.float32(0.0)
    hards = []
    for l in range(LEVELS):
        oh = jnp.concatenate([oh_all[l, s] for s in range(SEG)], axis=0)
        hard = _mm(bg_ref[l], oh, ((0,), (0,)))
        df = qs_[l] - hard
        loss = loss + jnp.sum(df * df)
        hards.append(hard)

    t = _mm(w_deq[2], hards[2], ((1,), (0,))) + b_deq[2] + b_side[2]
    y = _mm(w_res[2], t, ((1,), (0,))) + b_res[2]
    for l in (1, 0):
        t = (_mm(w_deq[l], hards[l], ((1,), (0,))) + b_deq[l]
             + _mm(w_side[l], y, ((1,), (0,))) + b_side[l])
        y = _mm(w_res[l], t, ((1,), (0,))) + b_res[l]
    out_ref[...] = y.T

    @pl.when(i == 0)
    def _init():
        loss_ref[0, 0] = loss

    @pl.when(i != 0)
    def _acc():
        loss_ref[0, 0] += loss


def kernel(heter_feature_2d, W_enc, b_enc, W_q, b_q, W_lat, b_lat,
           W_deq, b_deq, W_res, b_res, W_side, b_side, codebooks):
    Bq, C, Hq, Wq_ = heter_feature_2d.shape
    n = Bq * Hq * Wq_
    x = heter_feature_2d.reshape(C, n)
    blk = 4096
    grid = n // blk

    eye = jnp.eye(SEG, dtype=jnp.float32)
    cbT = jnp.transpose(codebooks, (0, 1, 3, 2))
    bd = jnp.einsum('lsdk,st->ltdsk', cbT, eye).reshape(LEVELS, C, KT)
    bg = jnp.transpose(bd, (0, 2, 1))
    csq = jnp.sum(codebooks * codebooks, axis=-1).reshape(LEVELS, KT, 1)
    col = lambda b: b.reshape(LEVELS, C, 1)

    full = lambda shape: pl.BlockSpec(shape, lambda i: (0,) * len(shape))
    out, loss = pl.pallas_call(
        functools.partial(_body, blk=blk),
        grid=(grid,),
        in_specs=[
            pl.BlockSpec((C, blk), lambda i: (0, i)),
            full((LEVELS, C, C)), full((LEVELS, C, 1)),
            full((LEVELS, C, C)), full((LEVELS, C, 1)),
            full((LEVELS, C, C)), full((LEVELS, C, 1)),
            full((LEVELS, C, C)), full((LEVELS, C, 1)),
            full((LEVELS, C, C)), full((LEVELS, C, 1)),
            full((LEVELS, C, C)), full((LEVELS, C, 1)),
            full((LEVELS, C, KT)), full((LEVELS, KT, C)),
            full((LEVELS, KT, 1)),
        ],
        out_specs=[
            pl.BlockSpec((blk, C), lambda i: (i, 0)),
            pl.BlockSpec((1, 1), lambda i: (0, 0),
                         memory_space=pltpu.SMEM),
        ],
        out_shape=[
            jax.ShapeDtypeStruct((n, C), jnp.float32),
            jax.ShapeDtypeStruct((1, 1), jnp.float32),
        ],
    )(x, W_enc, col(b_enc), W_q, col(b_q), W_lat, col(b_lat),
      W_deq, col(b_deq), W_res, col(b_res), W_side, col(b_side),
      bd, bg, csq)

    restored = out.reshape(Bq, Hq, Wq_, C)
    codebook_loss = loss[0, 0] * jnp.float32(1.25) / jnp.float32(n * C)
    return (restored, codebook_loss)


# speedup vs baseline: 4.2968x; 1.1526x over previous
import functools

import jax
import jax.numpy as jnp
from jax.experimental import pallas as pl
from jax.experimental.pallas import tpu as pltpu

CHANNEL = 64
SEG = 4
K = 128
LEVELS = 3
D = CHANNEL // SEG
KT = SEG * K


def _mm(a, b, dims):
    return jax.lax.dot_general(
        a, b, (dims, ((), ())), preferred_element_type=jnp.float32)


def _body(x_ref, w_enc, b_enc, w_q, b_q, w_lat, b_lat, w_deq, b_deq,
          w_res, b_res, w_side, b_side, bd_ref, bg_ref, csq_ref,
          out_ref, loss_ref, *, blk):
    i = pl.program_id(0)
    cur = x_ref[...]
    iota_f = jax.lax.broadcasted_iota(
        jnp.int32, (K, blk), 0).astype(jnp.float32)

    qs_, crosses = [], []
    for l in range(LEVELS):
        z = _mm(w_enc[l], cur, ((1,), (0,))) + b_enc[l]
        q = _mm(w_q[l], z, ((1,), (0,))) + b_q[l]
        crosses.append(_mm(bd_ref[l], q, ((0,), (0,))))
        qs_.append(q)
        if l < LEVELS - 1:
            cur = _mm(w_lat[l], z, ((1,), (0,))) + b_lat[l]

    pipes = [(l, s) for l in range(LEVELS) for s in range(SEG)]
    ds_all, mn_all, masked_all, idx_all, oh_all = {}, {}, {}, {}, {}
    for l, s in pipes:
        ds_all[l, s] = crosses[l][K * s:K * (s + 1), :] \
            + csq_ref[l][K * s:K * (s + 1), :]
    for l, s in pipes:
        mn_all[l, s] = jnp.min(ds_all[l, s], axis=0, keepdims=True)
    for l, s in pipes:
        masked_all[l, s] = jnp.where(
            ds_all[l, s] == mn_all[l, s], iota_f, jnp.float32(K))
    for l, s in pipes:
        idx_all[l, s] = jnp.min(masked_all[l, s], axis=0, keepdims=True)
    for l, s in pipes:
        oh_all[l, s] = (iota_f == idx_all[l, s]).astype(jnp.float32)

    loss = jnp.float32(0.0)
    hards = []
    for l in range(LEVELS):
        oh = jnp.concatenate([oh_all[l, s] for s in range(SEG)], axis=0)
        hard = _mm(bg_ref[l], oh, ((0,), (0,)))
        df = qs_[l] - hard
        loss = loss + jnp.sum(df * df)
        hards.append(hard)

    t = _mm(w_deq[2], hards[2], ((1,), (0,))) + b_deq[2] + b_side[2]
    y = _mm(w_res[2], t, ((1,), (0,))) + b_res[2]
    for l in (1, 0):
        t = (_mm(w_deq[l], hards[l], ((1,), (0,))) + b_deq[l]
             + _mm(w_side[l], y, ((1,), (0,))) + b_side[l])
        y = _mm(w_res[l], t, ((1,), (0,))) + b_res[l]
    out_ref[...] = y.T

    @pl.when(i == 0)
    def _init():
        loss_ref[0, 0] = loss

    @pl.when(i != 0)
    def _acc():
        loss_ref[0, 0] += loss


def kernel(heter_feature_2d, W_enc, b_enc, W_q, b_q, W_lat, b_lat,
           W_deq, b_deq, W_res, b_res, W_side, b_side, codebooks):
    Bq, C, Hq, Wq_ = heter_feature_2d.shape
    n = Bq * Hq * Wq_
    x = heter_feature_2d.reshape(C, n)
    blk = 4096
    grid = n // blk

    eye = jnp.eye(SEG, dtype=jnp.float32)
    cbT = jnp.transpose(codebooks, (0, 1, 3, 2))
    bd = jnp.einsum('lsdk,st->ltdsk', cbT, eye).reshape(LEVELS, C, KT)
    bg = jnp.transpose(bd, (0, 2, 1))
    bd = bd * jnp.float32(-2.0)
    csq = jnp.sum(codebooks * codebooks, axis=-1).reshape(LEVELS, KT, 1)
    col = lambda b: b.reshape(LEVELS, C, 1)

    full = lambda shape: pl.BlockSpec(shape, lambda i: (0,) * len(shape))
    out, loss = pl.pallas_call(
        functools.partial(_body, blk=blk),
        grid=(grid,),
        in_specs=[
            pl.BlockSpec((C, blk), lambda i: (0, i)),
            full((LEVELS, C, C)), full((LEVELS, C, 1)),
            full((LEVELS, C, C)), full((LEVELS, C, 1)),
            full((LEVELS, C, C)), full((LEVELS, C, 1)),
            full((LEVELS, C, C)), full((LEVELS, C, 1)),
            full((LEVELS, C, C)), full((LEVELS, C, 1)),
            full((LEVELS, C, C)), full((LEVELS, C, 1)),
            full((LEVELS, C, KT)), full((LEVELS, KT, C)),
            full((LEVELS, KT, 1)),
        ],
        out_specs=[
            pl.BlockSpec((blk, C), lambda i: (i, 0)),
            pl.BlockSpec((1, 1), lambda i: (0, 0),
                         memory_space=pltpu.SMEM),
        ],
        out_shape=[
            jax.ShapeDtypeStruct((n, C), jnp.float32),
            jax.ShapeDtypeStruct((1, 1), jnp.float32),
        ],
    )(x, W_enc, col(b_enc), W_q, col(b_q), W_lat, col(b_lat),
      W_deq, col(b_deq), W_res, col(b_res), W_side, col(b_side),
      bd, bg, csq)

    restored = out.reshape(Bq, Hq, Wq_, C)
    codebook_loss = loss[0, 0] * jnp.float32(1.25) / jnp.float32(n * C)
    return (restored, codebook_loss)


# speedup vs baseline: 4.3078x; 1.0026x over previous
import functools

import jax
import jax.numpy as jnp
from jax.experimental import pallas as pl
from jax.experimental.pallas import tpu as pltpu

CHANNEL = 64
SEG = 4
K = 128
LEVELS = 3
D = CHANNEL // SEG
KT = SEG * K


def _mm(a, b, dims):
    return jax.lax.dot_general(
        a, b, (dims, ((), ())), preferred_element_type=jnp.float32)


def _body(x_ref, w_enc, b_enc, w_q, b_q, w_lat, b_lat, w_deq, b_deq,
          w_res, b_res, w_side, b_side, cb_ref,
          out_ref, loss_ref, bd_ref, bg_ref, csq_ref, *, blk):
    i = pl.program_id(0)

    @pl.when(i == 0)
    def _build():
        bd_ref[...] = jnp.zeros((LEVELS, CHANNEL, KT), jnp.float32)
        bg_ref[...] = jnp.zeros((LEVELS, KT, CHANNEL), jnp.float32)
        for l in range(LEVELS):
            for s in range(SEG):
                cb = cb_ref[l, s]
                bg_ref[l, K * s:K * (s + 1), D * s:D * (s + 1)] = cb
                bd_ref[l, D * s:D * (s + 1), K * s:K * (s + 1)] = \
                    jnp.float32(-2.0) * cb.T
                csq_ref[l, K * s:K * (s + 1), :] = \
                    jnp.sum(cb * cb, axis=1, keepdims=True)

    cur = x_ref[...]
    iota_f = jax.lax.broadcasted_iota(
        jnp.int32, (K, blk), 0).astype(jnp.float32)

    qs_, crosses = [], []
    for l in range(LEVELS):
        z = _mm(w_enc[l], cur, ((1,), (0,))) + b_enc[l]
        q = _mm(w_q[l], z, ((1,), (0,))) + b_q[l]
        crosses.append(_mm(bd_ref[l], q, ((0,), (0,))))
        qs_.append(q)
        if l < LEVELS - 1:
            cur = _mm(w_lat[l], z, ((1,), (0,))) + b_lat[l]

    pipes = [(l, s) for l in range(LEVELS) for s in range(SEG)]
    ds_all, mn_all, masked_all, idx_all, oh_all = {}, {}, {}, {}, {}
    for l, s in pipes:
        ds_all[l, s] = crosses[l][K * s:K * (s + 1), :] \
            + csq_ref[l][K * s:K * (s + 1), :]
    for l, s in pipes:
        mn_all[l, s] = jnp.min(ds_all[l, s], axis=0, keepdims=True)
    for l, s in pipes:
        masked_all[l, s] = jnp.where(
            ds_all[l, s] == mn_all[l, s], iota_f, jnp.float32(K))
    for l, s in pipes:
        idx_all[l, s] = jnp.min(masked_all[l, s], axis=0, keepdims=True)
    for l, s in pipes:
        oh_all[l, s] = (iota_f == idx_all[l, s]).astype(jnp.float32)

    loss = jnp.float32(0.0)
    hards = []
    for l in range(LEVELS):
        oh = jnp.concatenate([oh_all[l, s] for s in range(SEG)], axis=0)
        hard = _mm(bg_ref[l], oh, ((0,), (0,)))
        df = qs_[l] - hard
        loss = loss + jnp.sum(df * df)
        hards.append(hard)

    t = _mm(w_deq[2], hards[2], ((1,), (0,))) + b_deq[2] + b_side[2]
    y = _mm(w_res[2], t, ((1,), (0,))) + b_res[2]
    for l in (1, 0):
        t = (_mm(w_deq[l], hards[l], ((1,), (0,))) + b_deq[l]
             + _mm(w_side[l], y, ((1,), (0,))) + b_side[l])
        y = _mm(w_res[l], t, ((1,), (0,))) + b_res[l]
    out_ref[...] = y.T

    @pl.when(i == 0)
    def _init():
        loss_ref[0, 0] = loss

    @pl.when(i != 0)
    def _acc():
        loss_ref[0, 0] += loss


def kernel(heter_feature_2d, W_enc, b_enc, W_q, b_q, W_lat, b_lat,
           W_deq, b_deq, W_res, b_res, W_side, b_side, codebooks):
    Bq, C, Hq, Wq_ = heter_feature_2d.shape
    n = Bq * Hq * Wq_
    x = heter_feature_2d.reshape(C, n)
    blk = 4096
    grid = n // blk

    col = lambda b: b.reshape(LEVELS, C, 1)

    full = lambda shape: pl.BlockSpec(shape, lambda i: (0,) * len(shape))
    out, loss = pl.pallas_call(
        functools.partial(_body, blk=blk),
        grid=(grid,),
        in_specs=[
            pl.BlockSpec((C, blk), lambda i: (0, i)),
            full((LEVELS, C, C)), full((LEVELS, C, 1)),
            full((LEVELS, C, C)), full((LEVELS, C, 1)),
            full((LEVELS, C, C)), full((LEVELS, C, 1)),
            full((LEVELS, C, C)), full((LEVELS, C, 1)),
            full((LEVELS, C, C)), full((LEVELS, C, 1)),
            full((LEVELS, C, C)), full((LEVELS, C, 1)),
            full((LEVELS, SEG, K, D)),
        ],
        out_specs=[
            pl.BlockSpec((blk, C), lambda i: (i, 0)),
            pl.BlockSpec((1, 1), lambda i: (0, 0),
                         memory_space=pltpu.SMEM),
        ],
        out_shape=[
            jax.ShapeDtypeStruct((n, C), jnp.float32),
            jax.ShapeDtypeStruct((1, 1), jnp.float32),
        ],
        scratch_shapes=[
            pltpu.VMEM((LEVELS, C, KT), jnp.float32),
            pltpu.VMEM((LEVELS, KT, C), jnp.float32),
            pltpu.VMEM((LEVELS, KT, 1), jnp.float32),
        ],
    )(x, W_enc, col(b_enc), W_q, col(b_q), W_lat, col(b_lat),
      W_deq, col(b_deq), W_res, col(b_res), W_side, col(b_side),
      codebooks)

    restored = out.reshape(Bq, Hq, Wq_, C)
    codebook_loss = loss[0, 0] * jnp.float32(1.25) / jnp.float32(n * C)
    return (restored, codebook_loss)
